# baseline (speedup 1.0000x reference)
import jax
import jax.numpy as jnp
from jax.experimental import pallas as pl
from jax.experimental.pallas import tpu as pltpu

N = 262144
H = 256
E = 32
G = 1024
EPS = 1e-5

B = 8192
NB = N // B
SMAX = G + NB
SW = 64

_F32 = jnp.float32
_BF16 = jnp.bfloat16

_CDIMS = (((1,), (0,)), ((), ()))
_TDIMS = (((0,), (0,)), ((), ()))
_VMEM_LIMIT = 100 * 1024 * 1024


def _split_hi_lo(a):
    hi = a.astype(_BF16)
    lo = (a - hi.astype(_F32)).astype(_BF16)
    return hi, lo


def _dotg(a, b, dims):
    return jax.lax.dot_general(a, b, dimension_numbers=dims,
                               preferred_element_type=_F32)


def _sc_stack(scales, c0, c1):
    sc = (1.0 + scales) * (c0 + c1)
    sch, scl = _split_hi_lo(sc)
    return jnp.concatenate([sch, scl, sch], axis=0)


def _onehot_t(lsid_1b, r, width):
    iota_sub = jax.lax.broadcasted_iota(
        jnp.int32, (width,) + lsid_1b.shape[1:], 0)
    sid = jnp.broadcast_to(lsid_1b - r * SW, iota_sub.shape)
    return jnp.where(sid == iota_sub, 1.0, 0.0).astype(_BF16)


def _contrib_kernel(evt_ref, x_ref, acc_ref):
    i = pl.program_id(0)

    @pl.when(i == 0)
    def _():
        acc_ref[...] = jnp.zeros_like(acc_ref)

    xb = x_ref[...].astype(_BF16)
    c = _dotg(evt_ref[...], xb, _CDIMS)
    acc_ref[...] += c[:E] + c[E:]


def _stats_kernel(sstart, send, sg, sptr, x_ref, ev_ref, scales_ref,
                  contrib_ref, lsid_ref, sq_ref, cnt_ref, sqloc_ref):
    b = pl.program_id(0)

    @pl.when(b == 0)
    def _():
        sq_ref[...] = jnp.zeros_like(sq_ref)
        cnt_ref[...] = jnp.zeros_like(cnt_ref)

    scs = _sc_stack(scales_ref[...], contrib_ref[...], 0.0)
    mean = _dotg(ev_ref[...], scs, _CDIMS)
    out = x_ref[...] - mean
    sqb = (out * out).astype(_BF16)

    lsid = lsid_ref[0]
    s0 = sptr[b]
    nseg = sptr[b + 1] - s0
    rounds = jax.lax.div(nseg + (SW - 1), SW)

    def round_body(r, _):
        sbase = s0 + r * SW
        oh_t = _onehot_t(lsid, r, SW)
        sqloc_ref[...] = _dotg(oh_t, sqb, _CDIMS)
        rem = jnp.minimum(nseg - r * SW, SW)

        def seg_body(jj, _):
            sidx = sbase + jj
            g = sg[sidx]
            cntv = (send[sidx] - sstart[sidx]).astype(_F32)
            sq_ref[pl.ds(g, 1), :] += sqloc_ref[pl.ds(jj, 1), :]
            cnt_ref[pl.ds(g, 1), :] += jnp.full((1, 128), 1.0, _F32) * cntv
            return 0

        jax.lax.fori_loop(0, rem, seg_body, 0)
        return 0

    jax.lax.fori_loop(0, rounds, round_body, 0)


def _finalize_kernel(sq_ref, cnt_ref, contrib_ref, scales_ref, w_ref,
                     winv_ref, scs_ref):
    cnt128 = jnp.maximum(cnt_ref[...], 1.0)
    cnt = jnp.concatenate([cnt128, cnt128], axis=1)
    var = sq_ref[...] / cnt
    winv_ref[...] = w_ref[...] * jax.lax.rsqrt(var + EPS)
    scs_ref[...] = _sc_stack(scales_ref[...], contrib_ref[...], 0.0)


def _norm_kernel(sstart, send, sg, sptr, x_ref, ev_ref, oh_ref, scs_ref,
                 winv_ref, bias_ref, lsid_ref, y_ref, wloc_ref):
    b = pl.program_id(0)

    mean = _dotg(ev_ref[...], scs_ref[...], _CDIMS)
    out = x_ref[...] - mean

    s0 = sptr[b]
    nseg = sptr[b + 1] - s0
    rounds = jax.lax.div(nseg + (SW - 1), SW)
    bias = bias_ref[...]

    def gather_rows(sbase, rem):
        wloc_ref[...] = jnp.zeros_like(wloc_ref)

        def seg_body(jj, _):
            g = sg[sbase + jj]
            wloc_ref[pl.ds(jj, 1), :] = winv_ref[pl.ds(g, 1), :]
            return 0

        jax.lax.fori_loop(0, rem, seg_body, 0)

    def wl_stack():
        wlh, wll = _split_hi_lo(wloc_ref[...])
        return jnp.concatenate([wlh, wll], axis=0)

    gather_rows(s0, jnp.minimum(nseg, SW))
    rs0 = _dotg(oh_ref[...], wl_stack(), _CDIMS)
    y_ref[...] = out * rs0 + bias

    @pl.when(rounds > 1)
    def _():
        lsid = lsid_ref[0]

        def round_body(r, _):
            gather_rows(s0 + r * SW, jnp.minimum(nseg - r * SW, SW))
            oh2 = pltpu.repeat(_onehot_t(lsid, r, SW), 2, axis=0)
            y_ref[...] += out * _dotg(oh2, wl_stack(), _TDIMS)
            return 0

        jax.lax.fori_loop(1, rounds, round_body, 0)


def kernel(x, evectors, batch, weight, bias, ev_scales):
    bi = batch.astype(jnp.int32)

    first = jnp.concatenate([jnp.ones((1,), jnp.bool_), bi[1:] != bi[:-1]])
    first = first | ((jnp.arange(N, dtype=jnp.int32) % B) == 0)
    seg_start = jnp.nonzero(first, size=SMAX, fill_value=N)[0].astype(jnp.int32)
    nxt = jnp.concatenate([seg_start[1:], jnp.full((1,), N, jnp.int32)])
    blk_end = (seg_start // B + 1) * B
    seg_end = jnp.minimum(nxt, blk_end)
    seg_g = bi[jnp.minimum(seg_start, N - 1)]
    seg_ptr = jnp.searchsorted(
        seg_start, jnp.arange(NB + 1, dtype=jnp.int32) * B,
        side='left').astype(jnp.int32)
    sid_global = jnp.cumsum(first.astype(jnp.int32)) - 1
    lsid = sid_global - jnp.repeat(seg_ptr[:NB], B)
    lsid3 = lsid.astype(jnp.int32).reshape(NB, 1, B)
    ohp = (lsid[:, None] == jnp.arange(SW, dtype=jnp.int32)[None, :])
    oh_pre = jnp.tile(ohp.astype(_BF16), (1, 2))

    evh = evectors.astype(_BF16)
    evl = (evectors - evh.astype(_F32)).astype(_BF16)
    ev_hl = jnp.concatenate([evh, evh, evl], axis=1)
    evt2 = jnp.concatenate([evh.T, evl.T], axis=0)

    w2 = weight.reshape(1, H)
    b2 = bias.reshape(1, H)

    contrib = pl.pallas_call(
        _contrib_kernel,
        out_shape=jax.ShapeDtypeStruct((E, H), _F32),
        grid=(NB,),
        in_specs=[
            pl.BlockSpec((2 * E, B), lambda i: (0, i)),
            pl.BlockSpec((B, H), lambda i: (i, 0)),
        ],
        out_specs=pl.BlockSpec((E, H), lambda i: (0, 0)),
        compiler_params=pltpu.CompilerParams(
            dimension_semantics=("arbitrary",),
            vmem_limit_bytes=_VMEM_LIMIT),
        name="gn2_contrib",
    )(evt2, x)

    sq, cnt = pl.pallas_call(
        _stats_kernel,
        out_shape=(
            jax.ShapeDtypeStruct((G, H), _F32),
            jax.ShapeDtypeStruct((G, 128), _F32),
        ),
        grid_spec=pltpu.PrefetchScalarGridSpec(
            num_scalar_prefetch=4,
            grid=(NB,),
            in_specs=[
                pl.BlockSpec((B, H), lambda i, *_: (i, 0)),
                pl.BlockSpec((B, 3 * E), lambda i, *_: (i, 0)),
                pl.BlockSpec((E, H), lambda i, *_: (0, 0)),
                pl.BlockSpec((E, H), lambda i, *_: (0, 0)),
                pl.BlockSpec((1, 1, B), lambda i, *_: (i, 0, 0)),
            ],
            out_specs=(
                pl.BlockSpec((G, H), lambda i, *_: (0, 0)),
                pl.BlockSpec((G, 128), lambda i, *_: (0, 0)),
            ),
            scratch_shapes=[pltpu.VMEM((SW, H), _F32)],
        ),
        compiler_params=pltpu.CompilerParams(
            dimension_semantics=("arbitrary",),
            vmem_limit_bytes=_VMEM_LIMIT),
        name="gn2_stats",
    )(seg_start, seg_end, seg_g, seg_ptr, x, ev_hl, ev_scales, contrib, lsid3)

    winv, scs = pl.pallas_call(
        _finalize_kernel,
        out_shape=(
            jax.ShapeDtypeStruct((G, H), _F32),
            jax.ShapeDtypeStruct((3 * E, H), _BF16),
        ),
        name="gn2_finalize",
    )(sq, cnt, contrib, ev_scales, w2)

    y = pl.pallas_call(
        _norm_kernel,
        out_shape=jax.ShapeDtypeStruct((N, H), _F32),
        grid_spec=pltpu.PrefetchScalarGridSpec(
            num_scalar_prefetch=4,
            grid=(NB,),
            in_specs=[
                pl.BlockSpec((B, H), lambda i, *_: (i, 0)),
                pl.BlockSpec((B, 3 * E), lambda i, *_: (i, 0)),
                pl.BlockSpec((B, 2 * SW), lambda i, *_: (i, 0)),
                pl.BlockSpec((3 * E, H), lambda i, *_: (0, 0)),
                pl.BlockSpec((G, H), lambda i, *_: (0, 0)),
                pl.BlockSpec((1, H), lambda i, *_: (0, 0)),
                pl.BlockSpec((1, 1, B), lambda i, *_: (i, 0, 0)),
            ],
            out_specs=pl.BlockSpec((B, H), lambda i, *_: (i, 0)),
            scratch_shapes=[pltpu.VMEM((SW, H), _F32)],
        ),
        compiler_params=pltpu.CompilerParams(
            dimension_semantics=("arbitrary",),
            vmem_limit_bytes=_VMEM_LIMIT),
        name="gn2_norm",
    )(seg_start, seg_end, seg_g, seg_ptr, x, ev_hl, oh_pre, scs, winv, b2,
      lsid3)

    return y
